# baseline (speedup 1.0000x reference)
import functools

import jax
import jax.numpy as jnp
from jax import lax
from jax.experimental import pallas as pl
from jax.experimental.pallas import tpu as pltpu
from jax.experimental.pallas import tpu_sc as plsc

_N_BLK = 2048
_GRP = 8


def _make_sc_gather(G, D, B):
    info = plsc.get_sparse_core_info()
    nc, ns = info.num_cores, info.num_subcores
    nw = nc * ns
    assert B % (8 * nw) == 0 and D % info.num_lanes == 0
    b_per_w = B // nw
    n_chunks = b_per_w // 16
    mesh = plsc.VectorSubcoreMesh(core_axis_name="c", subcore_axis_name="s")

    @functools.partial(
        pl.kernel,
        mesh=mesh,
        out_type=jax.ShapeDtypeStruct((B, D), jnp.float32),
        compiler_params=pltpu.CompilerParams(needs_layout_passes=False),
        scratch_types=[
            pltpu.VMEM((b_per_w,), jnp.int32),
            pltpu.VMEM((b_per_w,), jnp.int32),
            pltpu.VMEM((b_per_w, _GRP * D), jnp.float32),
            pltpu.VMEM((b_per_w, D), jnp.float32),
            pltpu.SemaphoreType.DMA,
        ],
    )
    def gather_kernel(table_hbm, idx_hbm, out_hbm, idx_v, gidx_v, grp_v,
                      out_v, sem):
        wid = lax.axis_index("s") * nc + lax.axis_index("c")
        base = wid * b_per_w
        pltpu.sync_copy(idx_hbm.at[pl.ds(base, b_per_w)], idx_v)
        iota16 = lax.iota(jnp.int32, 16)
        for k in range(n_chunks):
            iv = idx_v[pl.ds(k * 16, 16)]
            gidx_v[pl.ds(k * 16, 16)] = lax.shift_right_logical(iv, 3)
        pltpu.async_copy(table_hbm.at[gidx_v], grp_v, sem).wait()
        for k in range(n_chunks):
            iv = idx_v[pl.ds(k * 16, 16)]
            colbase = (iv & 7) * D
            rowids = iota16 + (k * 16)
            for j in range(D):
                vals = plsc.load_gather(grp_v, [rowids, colbase + j])
                jvec = jnp.full((16,), j, jnp.int32)
                plsc.store_scatter(out_v, [rowids, jvec], vals)
        pltpu.sync_copy(out_v, out_hbm.at[pl.ds(base, b_per_w)])

    return gather_kernel


def _mlp_body(flat_ref, w1_ref, b1_ref, w2_ref, b2_ref, out_ref, h_ref):
    @pl.when(pl.program_id(0) == 0)
    def _():
        h = lax.dot_general(
            flat_ref[...], w1_ref[...], (((1,), (1,)), ((), ())),
            preferred_element_type=jnp.float32)
        h_ref[...] = jnp.maximum(h + b1_ref[...], 0.0)

    out_ref[...] = lax.dot_general(
        h_ref[...], w2_ref[...], (((1,), (1,)), ((), ())),
        preferred_element_type=jnp.float32) + b2_ref[...]


def kernel(x, embed, W1, b1, W2, b2):
    batch, fan_in = x.shape
    vocab, hidden = W2.shape
    emb_dim = embed.shape[1]

    idx = x.reshape(-1).astype(jnp.int32)
    n_rows = embed.shape[0]
    pad = (-n_rows) % _GRP
    table_g = jnp.pad(embed, ((0, pad), (0, 0))).reshape(-1, _GRP * emb_dim)

    gather = _make_sc_gather(table_g.shape[0], emb_dim, idx.shape[0])
    rows = gather(table_g, idx)
    flat = rows.reshape(batch, fan_in * emb_dim)

    grid = pl.cdiv(vocab, _N_BLK)
    out = pl.pallas_call(
        _mlp_body,
        grid=(grid,),
        in_specs=[
            pl.BlockSpec((batch, fan_in * emb_dim), lambda i: (0, 0)),
            pl.BlockSpec(W1.shape, lambda i: (0, 0)),
            pl.BlockSpec((1, hidden), lambda i: (0, 0)),
            pl.BlockSpec((_N_BLK, hidden), lambda i: (i, 0)),
            pl.BlockSpec((1, _N_BLK), lambda i: (0, i)),
        ],
        out_specs=pl.BlockSpec((batch, _N_BLK), lambda i: (0, i)),
        out_shape=jax.ShapeDtypeStruct((batch, vocab), jnp.float32),
        scratch_shapes=[pltpu.VMEM((batch, hidden), jnp.float32)],
    )(flat, W1, b1.reshape(1, -1), W2, b2.reshape(1, -1))
    return out


# speedup vs baseline: 2.9770x; 2.9770x over previous
import functools

import jax
import jax.numpy as jnp
from jax import lax
from jax.experimental import pallas as pl
from jax.experimental.pallas import tpu as pltpu
from jax.experimental.pallas import tpu_sc as plsc

_N_BLK = 2048
_GRP = 8


def _make_sc_gather(G, D, B):
    info = plsc.get_sparse_core_info()
    nc, ns = info.num_cores, info.num_subcores
    nw = nc * ns
    assert B % (8 * nw) == 0 and D % info.num_lanes == 0
    b_per_w = B // nw
    n_chunks = b_per_w // 16
    mesh = plsc.VectorSubcoreMesh(core_axis_name="c", subcore_axis_name="s")

    @functools.partial(
        pl.kernel,
        mesh=mesh,
        out_type=jax.ShapeDtypeStruct((B, D), jnp.float32),
        compiler_params=pltpu.CompilerParams(needs_layout_passes=False),
        scratch_types=[
            pltpu.VMEM((b_per_w,), jnp.int32),
            pltpu.VMEM((b_per_w,), jnp.int32),
            pltpu.VMEM((b_per_w, _GRP * D), jnp.float32),
            pltpu.VMEM((b_per_w, D), jnp.float32),
            pltpu.SemaphoreType.DMA,
        ],
    )
    def gather_kernel(table_hbm, idx_hbm, out_hbm, idx_v, gidx_v, grp_v,
                      out_v, sem):
        wid = lax.axis_index("s") * nc + lax.axis_index("c")
        base = wid * b_per_w
        pltpu.sync_copy(idx_hbm.at[pl.ds(base, b_per_w)], idx_v)
        iota16 = lax.iota(jnp.int32, 16)
        for k in range(n_chunks):
            iv = idx_v[pl.ds(k * 16, 16)]
            gidx_v[pl.ds(k * 16, 16)] = lax.shift_right_logical(iv, 3)
        pltpu.async_copy(table_hbm.at[gidx_v], grp_v, sem).wait()
        for k in range(n_chunks):
            iv = idx_v[pl.ds(k * 16, 16)]
            colbase = (iv & 7) * D
            rowids = iota16 + (k * 16)
            for j in range(D):
                vals = plsc.load_gather(grp_v, [rowids, colbase + j])
                jvec = jnp.full((16,), j, jnp.int32)
                plsc.store_scatter(out_v, [rowids, jvec], vals)
        pltpu.sync_copy(out_v, out_hbm.at[pl.ds(base, b_per_w)])

    return gather_kernel


def _mlp_body(flat_ref, w1_ref, b1_ref, w2t_ref, b2_ref, out_ref, h_ref):
    @pl.when(pl.program_id(0) == 0)
    def _():
        h = lax.dot_general(
            flat_ref[...], w1_ref[...], (((1,), (1,)), ((), ())),
            preferred_element_type=jnp.float32)
        h_ref[...] = jnp.maximum(h + b1_ref[...], 0.0)

    acc = lax.dot_general(
        w2t_ref[...], h_ref[...], (((0,), (1,)), ((), ())),
        preferred_element_type=jnp.float32)
    bias = lax.dot_general(
        b2_ref[...], jnp.ones((1, h_ref.shape[0]), jnp.float32),
        (((0,), (0,)), ((), ())), preferred_element_type=jnp.float32)
    out_ref[...] = acc + bias


def kernel(x, embed, W1, b1, W2, b2):
    batch, fan_in = x.shape
    vocab, hidden = W2.shape
    emb_dim = embed.shape[1]

    idx = x.reshape(-1).astype(jnp.int32)
    flat_table = embed.reshape(-1)
    pad = (-flat_table.shape[0]) % (_GRP * emb_dim)
    table_g = jnp.pad(flat_table, (0, pad)).reshape(-1, _GRP * emb_dim)

    gather = _make_sc_gather(table_g.shape[0], emb_dim, idx.shape[0])
    rows = gather(table_g, idx)
    flat = rows.reshape(batch, fan_in * emb_dim)

    grid = pl.cdiv(vocab, _N_BLK)
    out_t = pl.pallas_call(
        _mlp_body,
        grid=(grid,),
        in_specs=[
            pl.BlockSpec((batch, fan_in * emb_dim), lambda i: (0, 0)),
            pl.BlockSpec(W1.shape, lambda i: (0, 0)),
            pl.BlockSpec((1, hidden), lambda i: (0, 0)),
            pl.BlockSpec((hidden, _N_BLK), lambda i: (0, i)),
            pl.BlockSpec((1, _N_BLK), lambda i: (0, i)),
        ],
        out_specs=pl.BlockSpec((_N_BLK, batch), lambda i: (i, 0)),
        out_shape=jax.ShapeDtypeStruct((vocab, batch), jnp.float32),
        scratch_shapes=[pltpu.VMEM((batch, hidden), jnp.float32)],
    )(flat, W1, b1.reshape(1, -1), W2.T, b2.reshape(1, -1))
    return out_t.T
